# baseline (speedup 1.0000x reference)
import functools

import jax
import jax.numpy as jnp
from jax.experimental import pallas as pl
from jax.experimental.pallas import tpu as pltpu

K_HOPS = 10
ADJ_SCALE = float(2 ** 18)
BM = 1024
BM_PREP = 256


def _prep_kernel(gamma_ref, s_ref, a_ref, xt_ref, q_ref, c18_ref, y0_ref,
                 x8_scr, *, bm, n, valid_last):
    i = pl.program_id(0)

    @pl.when(i == 0)
    def _init():
        x8_scr[...] = (xt_ref[...] * s_ref[0]).astype(jnp.float8_e4m3fn)

    q = (a_ref[...] * ADJ_SCALE).astype(jnp.float8_e4m3fn)
    q_ref[...] = q
    raw = jax.lax.dot_general(
        x8_scr[:, pl.ds(0, n)], q,
        (((1,), (1,)), ((), ())),
        preferred_element_type=jnp.float32,
    )
    c18_ref[...] = (raw * (1.0 / ADJ_SCALE)).astype(jnp.float8_e4m3fn)
    cols = pl.ds(i * bm, bm)
    y0_ref[...] = (gamma_ref[0] * xt_ref[:, cols]
                   + (gamma_ref[1] / (ADJ_SCALE * s_ref[0])) * raw)
    if valid_last < bm:
        @pl.when(i == pl.num_programs(0) - 1)
        def _zero_tail():
            q_ref[pl.ds(valid_last, bm - valid_last), :] = jnp.zeros(
                (bm - valid_last, n), jnp.float8_e4m3fn)


def _gpr_kernel(gamma_ref, s_ref, a_ref, c18_ref, y0_ref, o_ref,
                conv8_scr, *, bm, n, k_hops):
    k = pl.program_id(0)
    i = pl.program_id(1)

    @pl.when((k == 0) & (i == 0))
    def _init():
        conv8_scr[0] = c18_ref[...]
        o_ref[...] = y0_ref[...]

    p = k % 2
    raw = jax.lax.dot_general(
        conv8_scr[p, :, pl.ds(0, n)], a_ref[...],
        (((1,), (1,)), ((), ())),
        preferred_element_type=jnp.float32,
    )
    cols = pl.ds(i * bm, bm)
    conv8_scr[1 - p, :, cols] = (raw * (1.0 / ADJ_SCALE)).astype(
        jnp.float8_e4m3fn)
    o_ref[:, cols] = o_ref[:, cols] + (
        gamma_ref[k + 2] / (ADJ_SCALE * s_ref[0])) * raw


def kernel(x, adj, gamma):
    n, d = x.shape
    npad = -(-n // BM) * BM
    nb = npad // BM
    nbp = npad // BM_PREP
    valid_last = n - (nbp - 1) * BM_PREP

    xt = jnp.pad(x.T, ((0, 0), (0, npad - n)))
    s = (128.0 / jnp.maximum(jnp.max(jnp.abs(x)), 1e-30)).reshape(1)

    adj_q, c18, y0 = pl.pallas_call(
        functools.partial(_prep_kernel, bm=BM_PREP, n=n,
                          valid_last=valid_last),
        grid=(nbp,),
        in_specs=[
            pl.BlockSpec(memory_space=pltpu.SMEM),
            pl.BlockSpec(memory_space=pltpu.SMEM),
            pl.BlockSpec((BM_PREP, n), lambda i: (i, 0)),
            pl.BlockSpec((d, npad), lambda i: (0, 0)),
        ],
        out_specs=[
            pl.BlockSpec((BM_PREP, n), lambda i: (i, 0)),
            pl.BlockSpec((d, BM_PREP), lambda i: (0, i)),
            pl.BlockSpec((d, BM_PREP), lambda i: (0, i)),
        ],
        out_shape=[
            jax.ShapeDtypeStruct((npad, n), jnp.float8_e4m3fn),
            jax.ShapeDtypeStruct((d, npad), jnp.float8_e4m3fn),
            jax.ShapeDtypeStruct((d, npad), jnp.float32),
        ],
        scratch_shapes=[
            pltpu.VMEM((d, npad), jnp.float8_e4m3fn),
        ],
        compiler_params=pltpu.CompilerParams(
            dimension_semantics=("arbitrary",),
        ),
    )(gamma, s, adj, xt)

    body = functools.partial(_gpr_kernel, bm=BM, n=n, k_hops=K_HOPS)
    yt = pl.pallas_call(
        body,
        grid=(K_HOPS - 1, nb),
        in_specs=[
            pl.BlockSpec(memory_space=pltpu.SMEM),
            pl.BlockSpec(memory_space=pltpu.SMEM),
            pl.BlockSpec((BM, n), lambda k, i: (i, 0)),
            pl.BlockSpec((d, npad), lambda k, i: (0, 0)),
            pl.BlockSpec((d, npad), lambda k, i: (0, 0)),
        ],
        out_specs=pl.BlockSpec((d, npad), lambda k, i: (0, 0)),
        out_shape=jax.ShapeDtypeStruct((d, npad), jnp.float32),
        scratch_shapes=[
            pltpu.VMEM((2, d, npad), jnp.float8_e4m3fn),
        ],
        compiler_params=pltpu.CompilerParams(
            dimension_semantics=("arbitrary", "arbitrary"),
        ),
    )(gamma, s, adj_q, c18, y0)
    return yt[:, :n].T


# speedup vs baseline: 1.0551x; 1.0551x over previous
import functools

import jax
import jax.numpy as jnp
from jax.experimental import pallas as pl
from jax.experimental.pallas import tpu as pltpu

K_HOPS = 10
ADJ_SCALE = float(2 ** 18)
BM = 2048
BM_PREP = 256


def _prep_kernel(gamma_ref, s_ref, a_ref, xt_ref, q_ref, z0_ref,
                 x8_scr, *, bm, n, valid_last, k_hops):
    i = pl.program_id(0)

    @pl.when(i == 0)
    def _init():
        x8_scr[...] = (xt_ref[...] * s_ref[0]).astype(jnp.float8_e4m3fn)

    q = (a_ref[...] * ADJ_SCALE).astype(jnp.float8_e4m3fn)
    q_ref[...] = q
    raw = jax.lax.dot_general(
        x8_scr[:, pl.ds(0, n)], q,
        (((1,), (1,)), ((), ())),
        preferred_element_type=jnp.float32,
    )
    cols = pl.ds(i * bm, bm)
    z0_ref[...] = (
        (gamma_ref[k_hops] * (1.0 / ADJ_SCALE)) * raw
        + (gamma_ref[k_hops - 1] * s_ref[0]) * xt_ref[:, cols]
    ).astype(jnp.float8_e4m3fn)
    if valid_last < bm:
        @pl.when(i == pl.num_programs(0) - 1)
        def _zero_tail():
            q_ref[pl.ds(valid_last, bm - valid_last), :] = jnp.zeros(
                (bm - valid_last, n), jnp.float8_e4m3fn)


def _gpr_kernel(gamma_ref, s_ref, a_ref, z8_ref, xt_ref, o_ref,
                z8_scr, *, bm, n, k_hops):
    k = pl.program_id(0)
    i = pl.program_id(1)

    @pl.when((k == 0) & (i == 0))
    def _init():
        z8_scr[0] = z8_ref[...]

    p = k % 2
    raw = jax.lax.dot_general(
        z8_scr[p, :, pl.ds(0, n)], a_ref[...],
        (((1,), (1,)), ((), ())),
        preferred_element_type=jnp.float32,
    )
    cols = pl.ds(i * bm, bm)
    j = k_hops - 2 - k

    @pl.when(k < k_hops - 2)
    def _step():
        z8_scr[1 - p, :, cols] = (
            (1.0 / ADJ_SCALE) * raw
            + (gamma_ref[j] * s_ref[0]) * xt_ref[:, cols]
        ).astype(jnp.float8_e4m3fn)

    @pl.when(k == k_hops - 2)
    def _emit():
        o_ref[...] = ((1.0 / (ADJ_SCALE * s_ref[0])) * raw
                      + gamma_ref[0] * xt_ref[:, cols])


def kernel(x, adj, gamma):
    n, d = x.shape
    npad = -(-n // BM) * BM
    nb = npad // BM
    nbp = npad // BM_PREP
    valid_last = n - (nbp - 1) * BM_PREP

    xt = jnp.pad(x.T, ((0, 0), (0, npad - n)))
    s = (128.0 / jnp.maximum(jnp.max(jnp.abs(x)), 1e-30)).reshape(1)

    adj_q, z8 = pl.pallas_call(
        functools.partial(_prep_kernel, bm=BM_PREP, n=n,
                          valid_last=valid_last, k_hops=K_HOPS),
        grid=(nbp,),
        in_specs=[
            pl.BlockSpec(memory_space=pltpu.SMEM),
            pl.BlockSpec(memory_space=pltpu.SMEM),
            pl.BlockSpec((BM_PREP, n), lambda i: (i, 0)),
            pl.BlockSpec((d, npad), lambda i: (0, 0)),
        ],
        out_specs=[
            pl.BlockSpec((BM_PREP, n), lambda i: (i, 0)),
            pl.BlockSpec((d, BM_PREP), lambda i: (0, i)),
        ],
        out_shape=[
            jax.ShapeDtypeStruct((npad, n), jnp.float8_e4m3fn),
            jax.ShapeDtypeStruct((d, npad), jnp.float8_e4m3fn),
        ],
        scratch_shapes=[
            pltpu.VMEM((d, npad), jnp.float8_e4m3fn),
        ],
        compiler_params=pltpu.CompilerParams(
            dimension_semantics=("arbitrary",),
        ),
    )(gamma, s, adj, xt)

    body = functools.partial(_gpr_kernel, bm=BM, n=n, k_hops=K_HOPS)
    yt = pl.pallas_call(
        body,
        grid=(K_HOPS - 1, nb),
        in_specs=[
            pl.BlockSpec(memory_space=pltpu.SMEM),
            pl.BlockSpec(memory_space=pltpu.SMEM),
            pl.BlockSpec((BM, n), lambda k, i: (i, 0)),
            pl.BlockSpec((d, npad), lambda k, i: (0, 0)),
            pl.BlockSpec((d, npad), lambda k, i: (0, 0)),
        ],
        out_specs=pl.BlockSpec(
            (d, BM), lambda k, i: (0, jnp.where(k == K_HOPS - 2, i, 0))
        ),
        out_shape=jax.ShapeDtypeStruct((d, npad), jnp.float32),
        scratch_shapes=[
            pltpu.VMEM((2, d, npad), jnp.float8_e4m3fn),
        ],
        compiler_params=pltpu.CompilerParams(
            dimension_semantics=("arbitrary", "arbitrary"),
        ),
    )(gamma, s, adj_q, z8, xt)
    return yt[:, :n].T
